# baseline (speedup 1.0000x reference)
import functools
import math

import jax
import jax.numpy as jnp
import numpy as np
from jax import lax
from jax.experimental import pallas as pl
from jax.experimental.pallas import tpu as pltpu
from jax.experimental.pallas import tpu_sc as plsc

_VOCAB = 1000000
_SIZE = 64
_BATCH = 4096
_SEQ = 200
_SCALE = float(math.sqrt(_SIZE))
_BLK = 128
_OUT_BLK = _BLK // 2


def _make_pe(seq, d):
    pos = np.arange(seq, dtype=np.float32)[:, None]
    div = np.exp(np.arange(0, d, 2, dtype=np.float32) * (-math.log(10000.0) / d))
    pe = np.zeros((seq, d), dtype=np.float32)
    pe[:, 0::2] = np.sin(pos * div)
    pe[:, 1::2] = np.cos(pos * div)
    return pe


_PE2 = np.tile(_make_pe(_SEQ, _SIZE).reshape(_SEQ // 2, 2 * _SIZE), (2, 1))


@functools.lru_cache(maxsize=1)
def _build():
    info = plsc.get_sparse_core_info()
    nc, ns = info.num_cores, info.num_subcores
    nw = nc * ns
    tokens = _BATCH * _SEQ
    per_w = tokens // nw
    n_blk = per_w // _BLK
    out_rows = tokens // 2

    mesh = plsc.VectorSubcoreMesh(core_axis_name="c", subcore_axis_name="s")

    @functools.partial(
        pl.kernel,
        mesh=mesh,
        out_type=jax.ShapeDtypeStruct((out_rows, 2 * _SIZE), jnp.float32),
        scratch_types=[
            pltpu.VMEM((_BLK,), jnp.int32),
            pltpu.VMEM((_BLK,), jnp.int32),
            pltpu.VMEM((_BLK,), jnp.int32),
            pltpu.VMEM((_BLK,), jnp.int32),
            pltpu.VMEM((_BLK, 2 * _SIZE), jnp.float32),
            pltpu.VMEM((_BLK, 2 * _SIZE), jnp.float32),
            pltpu.VMEM((_OUT_BLK, 2 * _SIZE), jnp.float32),
            pltpu.VMEM((_OUT_BLK, 2 * _SIZE), jnp.float32),
            pltpu.VMEM((2 * (_SEQ // 2), 2 * _SIZE), jnp.float32),
            pltpu.SemaphoreType.DMA,
            pltpu.SemaphoreType.DMA,
        ],
    )
    def k(xh_hbm, xb_hbm, table_hbm, pe_hbm, out_hbm,
          idx0, idx1, bit0, bit1, rows0, rows1, ob0, ob1, pe_v,
          sem_g, sem_w):
        wid = lax.axis_index("s") * nc + lax.axis_index("c")
        base = wid * per_w
        pltpu.sync_copy(pe_hbm, pe_v)

        idx_b = (idx0, idx1)
        bit_b = (bit0, bit1)
        rows_b = (rows0, rows1)
        out_b = (ob0, ob1)

        def fire(blk, par):
            off = base + blk * _BLK
            pltpu.sync_copy(xh_hbm.at[pl.ds(off, _BLK)], idx_b[par])
            pltpu.sync_copy(xb_hbm.at[pl.ds(off, _BLK)], bit_b[par])
            pltpu.async_copy(table_hbm.at[idx_b[par]], rows_b[par], sem_g)

        def wait_gather(par):
            pltpu.make_async_copy(
                out_hbm.at[pl.ds(0, _BLK)], rows_b[par], sem_g).wait()

        def wait_write(par):
            pltpu.make_async_copy(
                out_b[par], out_hbm.at[pl.ds(0, _OUT_BLK)], sem_w).wait()

        fire(0, 0)

        def body(i, carry):
            for par in range(2):
                blk = 2 * i + par

                if par == 1:
                    wait_write(1 - par)
                else:
                    @pl.when(blk > 0)
                    def _():
                        wait_write(1 - par)

                wait_gather(par)

                @pl.when(blk + 1 < n_blk)
                def _():
                    fire(blk + 1, 1 - par)

                rb = rows_b[par]
                bb = bit_b[par]
                ob = out_b[par]
                s0 = lax.rem(blk * _OUT_BLK, _SEQ // 2)

                @plsc.parallel_loop(0, _OUT_BLK, step=1, unroll=2)
                def _(r):
                    pr = s0 + r
                    for h in range(2):
                        t = 2 * r + h
                        for c in range(_SIZE // 16):
                            lo = rb[t, pl.ds(c * 16, 16)]
                            so = pl.ds(h * _SIZE + c * 16, 16)
                            ob[r, so] = lo * _SCALE + pe_v[pr, so]

                off2 = wid * (per_w // 2) + blk * _OUT_BLK
                pltpu.async_copy(ob, out_hbm.at[pl.ds(off2, _OUT_BLK)], sem_w)
            return carry

        lax.fori_loop(0, n_blk // 2, body, 0)
        wait_write(1)

    return k


def kernel(x, emb_table):
    xf = x.reshape(-1)
    xh = jnp.right_shift(xf, 1)
    xb = jnp.bitwise_and(xf, 1)
    t2 = emb_table.reshape(_VOCAB // 2, 2 * _SIZE)
    out = _build()(xh, xb, t2, jnp.asarray(_PE2))
    return out.reshape(_BATCH, _SEQ, _SIZE)


# speedup vs baseline: 1.1833x; 1.1833x over previous
import functools
import math

import jax
import jax.numpy as jnp
import numpy as np
from jax import lax
from jax.experimental import pallas as pl
from jax.experimental.pallas import tpu as pltpu
from jax.experimental.pallas import tpu_sc as plsc

_VOCAB = 1000000
_SIZE = 64
_BATCH = 4096
_SEQ = 200
_SCALE = float(math.sqrt(_SIZE))


def _make_pe(seq, d):
    pos = np.arange(seq, dtype=np.float32)[:, None]
    div = np.exp(np.arange(0, d, 2, dtype=np.float32) * (-math.log(10000.0) / d))
    pe = np.zeros((seq, d), dtype=np.float32)
    pe[:, 0::2] = np.sin(pos * div)
    pe[:, 1::2] = np.cos(pos * div)
    return pe


_PE = _make_pe(_SEQ, _SIZE)


@functools.lru_cache(maxsize=1)
def _build():
    info = plsc.get_sparse_core_info()
    nc, ns = info.num_cores, info.num_subcores
    nw = nc * ns
    rows_total = _BATCH * _SEQ
    per_w = rows_total // nw
    n_blk = per_w // _SEQ
    half_seq = _SEQ // 2
    split = 128
    rest = _SEQ - split

    mesh = plsc.VectorSubcoreMesh(core_axis_name="c", subcore_axis_name="s")

    @functools.partial(
        pl.kernel,
        mesh=mesh,
        compiler_params=pltpu.CompilerParams(use_tc_tiling_on_sc=False),
        out_type=jax.ShapeDtypeStruct((rows_total // 2, 2 * _SIZE), jnp.float32),
        scratch_types=[
            pltpu.VMEM((_SEQ,), jnp.int32),
            pltpu.VMEM((_SEQ,), jnp.int32),
            pltpu.VMEM((_SEQ, _SIZE), jnp.float32),
            pltpu.VMEM((_SEQ, _SIZE), jnp.float32),
            pltpu.VMEM((_SEQ // 2, 2 * _SIZE), jnp.float32),
            pltpu.VMEM((_SEQ // 2, 2 * _SIZE), jnp.float32),
            pltpu.VMEM((_SEQ // 2, 2 * _SIZE), jnp.float32),
            pltpu.SemaphoreType.DMA,
            pltpu.SemaphoreType.DMA,
        ],
    )
    def k(xf_hbm, table_hbm, pe_hbm, out_hbm,
          idx0, idx1, rows0, rows1, ob0, ob1, pe_v, sem_g, sem_w):
        wid = lax.axis_index("s") * nc + lax.axis_index("c")
        base = wid * per_w
        pltpu.sync_copy(pe_hbm, pe_v)

        idx_b = (idx0, idx1)
        rows_b = (rows0, rows1)
        out_b = (ob0, ob1)

        def fire(blk, par):
            ib, rb = idx_b[par], rows_b[par]
            off = base + blk * _SEQ
            pltpu.sync_copy(xf_hbm.at[pl.ds(off, _SEQ)], ib)
            pltpu.async_copy(
                table_hbm.at[ib.at[pl.ds(0, split)]],
                rb.at[pl.ds(0, split)], sem_g)
            pltpu.async_copy(
                table_hbm.at[ib.at[pl.ds(split, rest)]],
                rb.at[pl.ds(split, rest)], sem_g)

        def wait_gather(par):
            pltpu.make_async_copy(
                out_hbm.at[pl.ds(0, half_seq)], rows_b[par], sem_g).wait()

        def wait_write(par):
            pltpu.make_async_copy(
                out_b[par], out_hbm.at[pl.ds(0, half_seq)], sem_w).wait()

        fire(0, 0)

        def body(i, carry):
            for par in range(2):
                blk = 2 * i + par

                if par == 1:
                    wait_write(1 - par)
                else:
                    @pl.when(blk > 0)
                    def _():
                        wait_write(1 - par)

                wait_gather(par)

                @pl.when(blk + 1 < n_blk)
                def _():
                    fire(blk + 1, 1 - par)

                rb = rows_b[par]
                ob = out_b[par]

                @plsc.parallel_loop(0, half_seq, step=1, unroll=4)
                def _(rp):
                    for h in range(2):
                        for c in range(_SIZE // 16):
                            so = pl.ds(h * _SIZE + c * 16, 16)
                            ob[rp, so] = (
                                rb[2 * rp + h, pl.ds(c * 16, 16)] * _SCALE
                                + pe_v[rp, so])

                off2 = wid * (per_w // 2) + blk * half_seq
                pltpu.async_copy(ob, out_hbm.at[pl.ds(off2, half_seq)], sem_w)
            return carry

        lax.fori_loop(0, n_blk // 2, body, 0)
        wait_write(1)

    return k


def kernel(x, emb_table):
    xf = x.reshape(-1)
    pe2 = jnp.asarray(_PE).reshape(_SEQ // 2, 2 * _SIZE)
    out = _build()(xf, emb_table, pe2)
    return out.reshape(_BATCH, _SEQ, _SIZE)
